# baseline (speedup 1.0000x reference)
import functools

import jax
import jax.numpy as jnp
from jax import lax
from jax.experimental import pallas as pl
from jax.experimental.pallas import tpu as pltpu
from jax.experimental.pallas import tpu_sc as plsc

_D = 64
_CHUNK = 128
_NC = 2
_NS = 16
_NW = _NC * _NS


@functools.lru_cache(maxsize=None)
def _make_kernel(n_total):
    n_per_w = n_total // _NW
    n_chunks = n_per_w // _CHUNK
    mesh = plsc.VectorSubcoreMesh(core_axis_name="c", subcore_axis_name="s")

    @functools.partial(
        pl.kernel,
        out_type=jax.ShapeDtypeStruct((n_total, _D), jnp.float32),
        mesh=mesh,
        scratch_types=[
            pltpu.VMEM((n_chunks, _CHUNK), jnp.int32),
            pltpu.VMEM((_CHUNK, _D), jnp.float32),
            pltpu.SemaphoreType.DMA,
        ],
        compiler_params=pltpu.CompilerParams(use_tc_tiling_on_sc=False),
    )
    def body(idx_hbm, table_hbm, out_hbm, idx_v, rows_v, sem):
        wid = lax.axis_index("s") * _NC + lax.axis_index("c")
        pltpu.sync_copy(idx_hbm.at[wid], idx_v)
        base = wid * n_per_w

        def step(j, carry):
            pltpu.async_copy(table_hbm.at[idx_v.at[j]], rows_v, sem).wait()
            pltpu.sync_copy(
                rows_v, out_hbm.at[pl.ds(base + j * _CHUNK, _CHUNK)]
            )
            return carry

        lax.fori_loop(0, n_chunks, step, 0)

    return body


def kernel(words, table):
    b, h = words.shape
    n_total = b * h
    idx = words.reshape(_NW, n_total // _NW // _CHUNK, _CHUNK)
    out = _make_kernel(n_total)(idx, table)
    return out.reshape(b, h, _D)


# speedup vs baseline: 1.1201x; 1.1201x over previous
import functools

import jax
import jax.numpy as jnp
from jax import lax
from jax.experimental import pallas as pl
from jax.experimental.pallas import tpu as pltpu
from jax.experimental.pallas import tpu_sc as plsc

_D = 64
_CHUNK = 128
_NC = 2
_NS = 16
_NW = _NC * _NS
_NBUF = 8
_AHEAD = 4


@functools.lru_cache(maxsize=None)
def _make_kernel(n_total):
    n_per_w = n_total // _NW
    n_chunks = n_per_w // _CHUNK
    assert n_chunks % _NBUF == 0
    mesh = plsc.VectorSubcoreMesh(core_axis_name="c", subcore_axis_name="s")

    @functools.partial(
        pl.kernel,
        out_type=jax.ShapeDtypeStruct((n_total, _D), jnp.float32),
        mesh=mesh,
        scratch_types=[
            pltpu.VMEM((n_chunks, _CHUNK), jnp.int32),
            pltpu.VMEM((_NBUF, _CHUNK, _D), jnp.float32),
            pltpu.SemaphoreType.DMA((_NBUF,)),
            pltpu.SemaphoreType.DMA((_NBUF,)),
        ],
        compiler_params=pltpu.CompilerParams(use_tc_tiling_on_sc=False),
    )
    def body(idx_hbm, table_hbm, out_hbm, idx_v, rows_v, g_sem, o_sem):
        wid = lax.axis_index("s") * _NC + lax.axis_index("c")
        pltpu.sync_copy(idx_hbm.at[wid], idx_v)
        base = wid * n_per_w

        def start_gather(j, slot):
            pltpu.async_copy(
                table_hbm.at[idx_v.at[j]], rows_v.at[slot], g_sem.at[slot]
            )

        for b in range(_AHEAD):
            start_gather(b, b)

        def group(g, carry):
            for b in range(_NBUF):
                j = g * _NBUF + b
                j2 = j + _AHEAD
                s2 = (b + _AHEAD) % _NBUF

                @pl.when(jnp.logical_and(j2 < n_chunks, j2 >= _NBUF))
                def _():
                    pltpu.make_async_copy(
                        rows_v.at[s2],
                        out_hbm.at[pl.ds(base, _CHUNK)],
                        o_sem.at[s2],
                    ).wait()

                @pl.when(j2 < n_chunks)
                def _():
                    start_gather(j2, s2)

                pltpu.make_async_copy(
                    table_hbm.at[idx_v.at[j]],
                    rows_v.at[b],
                    g_sem.at[b],
                ).wait()
                pltpu.async_copy(
                    rows_v.at[b],
                    out_hbm.at[pl.ds(base + j * _CHUNK, _CHUNK)],
                    o_sem.at[b],
                )
            return carry

        lax.fori_loop(0, n_chunks // _NBUF, group, 0)

        for b in range(_NBUF):
            pltpu.make_async_copy(
                rows_v.at[b], out_hbm.at[pl.ds(base, _CHUNK)], o_sem.at[b]
            ).wait()

    return body


def kernel(words, table):
    b, h = words.shape
    n_total = b * h
    idx = words.reshape(_NW, n_total // _NW // _CHUNK, _CHUNK)
    out = _make_kernel(n_total)(idx, table)
    return out.reshape(b, h, _D)
